# baseline (speedup 1.0000x reference)
import jax
import jax.numpy as jnp
from jax.experimental import pallas as pl
from jax.experimental.pallas import tpu as pltpu

B, N, DIM, H, DH, NN = 4, 512, 256, 4, 32, 32
SCALE = 8.0
PB = 128
CQ = 128
NC = N // CQ


def _psi(x):
    return jnp.sign(x) * jnp.log1p(jnp.abs(x))


def _ln(x, g, b, eps=1e-5):
    m = jnp.mean(x, axis=-1, keepdims=True)
    v = jnp.mean((x - m) ** 2, axis=-1, keepdims=True)
    return (x - m) * jax.lax.rsqrt(v + eps) * g + b


def _silu(x):
    return x * jax.nn.sigmoid(x)


def _gelu(x):
    return 0.5 * x * (1.0 + jax.lax.erf(x * 0.7071067811865476))


def _lorentz_kernel(
    feats_ref, coors_ref, fc_ref, cc_ref, ct_ref,
    ng_ref, nb_ref, wqkv_ref, wout_ref, bout_ref,
    wc1_ref, wc2_ref, wg_ref, bg_ref, cns_ref, comb_ref,
    w0_ref, b0_ref, l0g_ref, l0b_ref,
    w1_ref, b1_ref, l1g_ref, l1b_ref,
    w2_ref, b2_ref, l2g_ref, l2b_ref,
    wqk_ref, bqk_ref, wv_ref, bv_ref,
    node_ref, cout_ref,
    ohs_ref, ds_ref,
):
    feats = feats_ref[0]
    coors = coors_ref[0]

    fn = _ln(feats, ng_ref[0], nb_ref[0])
    qkv = jnp.dot(fn, wqkv_ref[...], preferred_element_type=jnp.float32)

    r = jax.lax.broadcasted_iota(jnp.int32, (H * DH, H * DH), 0) // DH
    c = jax.lax.broadcasted_iota(jnp.int32, (H * DH, H * DH), 1) // DH
    S = (r == c).astype(jnp.float32)

    k = qkv[:, H * DH:2 * H * DH]
    v = qkv[:, 2 * H * DH:]
    ksq = jnp.dot(k * k, S, preferred_element_type=jnp.float32)
    kn = k / jnp.maximum(jnp.sqrt(ksq), 1e-12)
    kvc = jnp.concatenate([kn, v, coors], axis=1)

    fnc = _ln(fc_ref[0], ng_ref[0], nb_ref[0])
    q = jnp.dot(fnc, wqkv_ref[:, :H * DH], preferred_element_type=jnp.float32)
    qsq = jnp.dot(q * q, S, preferred_element_type=jnp.float32)
    qn = q / jnp.maximum(jnp.sqrt(qsq), 1e-12)

    coors_c = cc_ref[0]

    ct = ct_ref[0]
    raw = None
    for ci in range(4):
        dq = coors_c[:, ci:ci + 1] - ct[ci:ci + 1, :]
        sq = dq * dq
        raw = sq if ci == 0 else raw - sq
    dist = _psi(raw)

    iota_j = jax.lax.broadcasted_iota(jnp.int32, (CQ, N), 1)

    def body(t, d):
        dmin = jnp.min(d, axis=1, keepdims=True)
        ismin = d == dmin
        idx = jnp.min(jnp.where(ismin, iota_j, N), axis=1, keepdims=True)
        oh = iota_j == idx
        d = jnp.where(oh, jnp.float32(1e30), d)
        ohs_ref[t] = oh.astype(jnp.bfloat16)
        ds_ref[t] = dmin
        return d

    jax.lax.fori_loop(0, NN, body, dist)

    ohf = ohs_ref[...].reshape(NN * CQ, N)
    kvc_hi = kvc.astype(jnp.bfloat16)
    kvc_lo = (kvc - kvc_hi.astype(jnp.float32)).astype(jnp.bfloat16)
    g = (jnp.dot(ohf, kvc_hi, preferred_element_type=jnp.float32)
         + jnp.dot(ohf, kvc_lo, preferred_element_type=jnp.float32))
    kg = g[:, :H * DH].reshape(NN, CQ, H * DH)
    vg = g[:, H * DH:2 * H * DH].reshape(NN, CQ, H * DH)
    cg = g[:, 2 * H * DH:2 * H * DH + 4].reshape(NN, CQ, 4)

    w0v = w0_ref[...]
    b0v = b0_ref[...]
    g0 = l0g_ref[...]
    u = w0v - jnp.mean(w0v, axis=1, keepdims=True)
    ec = b0v - jnp.mean(b0v, axis=1, keepdims=True)
    v0 = jnp.mean(u * u, axis=1, keepdims=True).reshape(1, 1, 1)
    c0 = jnp.mean(u * ec, axis=1, keepdims=True).reshape(1, 1, 1)
    vb = jnp.mean(ec * ec, axis=1, keepdims=True).reshape(1, 1, 1)
    d0 = ds_ref[...]
    s = jax.lax.rsqrt(d0 * d0 * v0 + 2.0 * d0 * c0 + vb + 1e-5)
    x3 = (d0 * s) * (u * g0)[None] + s * (ec * g0)[None] + l0b_ref[...][None]
    x = _silu(x3.reshape(NN * CQ, PB))
    x = jnp.dot(x, w1_ref[...], preferred_element_type=jnp.float32) + b1_ref[...]
    x = _silu(_ln(x, l1g_ref[...], l1b_ref[...]))
    x = jnp.dot(x, w2_ref[...], preferred_element_type=jnp.float32) + b2_ref[...]
    x = _silu(_ln(x, l2g_ref[...], l2b_ref[...]))
    qk_pos = jnp.dot(x, wqk_ref[...], preferred_element_type=jnp.float32) + bqk_ref[...]
    vpos = jnp.dot(x, wv_ref[...], preferred_element_type=jnp.float32) + bv_ref[...]

    prod = (kg * qn[None]).reshape(NN * CQ, H * DH)
    qk_g = jnp.dot(prod, S, preferred_element_type=jnp.float32)
    qk = qk_g * SCALE + qk_pos
    qk3 = qk.reshape(NN, CQ, H * DH)

    mx = jnp.max(qk3, axis=0, keepdims=True)
    e = jnp.exp(qk3 - mx)
    attn = e / jnp.sum(e, axis=0, keepdims=True)

    vtot = vg + vpos.reshape(NN, CQ, H * DH)
    out_pre = jnp.sum(attn * vtot, axis=0)
    node = jnp.dot(out_pre, wout_ref[...], preferred_element_type=jnp.float32) + bout_ref[...]
    node_ref[0] = node

    h1 = _gelu(jnp.dot(qk, wc1_ref[...], preferred_element_type=jnp.float32))
    cw = jnp.dot(h1, wc2_ref[...], preferred_element_type=jnp.float32)
    cw3 = cw.reshape(NN, CQ, H * DH)
    cmx = jnp.max(cw3, axis=0, keepdims=True)
    ce = jnp.exp(cw3 - cmx)
    ca = ce / jnp.sum(ce, axis=0, keepdims=True)
    gate = jnp.tanh(
        jnp.dot(qk, wg_ref[...], preferred_element_type=jnp.float32) + bg_ref[...]
    ).reshape(NN, CQ, H * DH)
    w = jnp.sum(ca * gate * comb_ref[...], axis=2, keepdims=True)
    rel = (coors_c[None] - cg) * cns_ref[0, 0]
    cout_ref[0] = jnp.sum(w * rel, axis=0)


def kernel(feats, coors, params):
    p = params
    coors_t = jnp.swapaxes(coors, 1, 2)

    def rep(x, axis):
        return jnp.repeat(x, DH, axis=axis)

    wc1 = rep(p['W_c1'], 0) / DH
    wc2 = rep(p['W_c2'], 1)
    wg = rep(rep(p['W_gate'], 0), 1) / DH
    bg = rep(p['b_gate'][None], 1)
    comb = rep(p['coors_combine'][None], 1) / DH
    wqk = rep(p['pb_Wqk'], 1)
    bqk = rep(p['pb_bqk'][None], 1)

    def row(x):
        return x[None].astype(jnp.float32)

    inputs = [
        feats, coors, feats, coors, coors_t,
        row(p['norm_gamma']), row(p['norm_beta']),
        p['W_qkv'], p['W_out'], row(p['b_out']),
        wc1, wc2, wg, bg,
        p['coors_norm_scale'][None], comb,
        p['pb_W0'], row(p['pb_b0']), row(p['pb_ln0_g']), row(p['pb_ln0_b']),
        p['pb_W1'], row(p['pb_b1']), row(p['pb_ln1_g']), row(p['pb_ln1_b']),
        p['pb_W2'], row(p['pb_b2']), row(p['pb_ln2_g']), row(p['pb_ln2_b']),
        wqk, bqk,
        p['pb_Wv'], row(p['pb_bv']),
    ]

    def batch_spec(shape):
        return pl.BlockSpec((1,) + shape, lambda b, c: (b, 0, 0))

    def full_spec(x):
        nd = x.ndim
        return pl.BlockSpec(x.shape, lambda b, c, _n=nd: (0,) * _n)

    def chunk_spec(lanes):
        return pl.BlockSpec((1, CQ, lanes), lambda b, c: (b, c, 0))

    in_specs = [
        batch_spec((N, DIM)), batch_spec((N, 4)),
        chunk_spec(DIM), chunk_spec(4),
        batch_spec((4, N)),
    ] + [full_spec(x) for x in inputs[5:]]

    out_shape = [
        jax.ShapeDtypeStruct((B, N, DIM), jnp.float32),
        jax.ShapeDtypeStruct((B, N, 4), jnp.float32),
    ]
    out_specs = [
        pl.BlockSpec((1, CQ, DIM), lambda b, c: (b, c, 0)),
        pl.BlockSpec((1, CQ, 4), lambda b, c: (b, c, 0)),
    ]

    node_out, coors_out = pl.pallas_call(
        _lorentz_kernel,
        grid=(B, NC),
        in_specs=in_specs,
        out_specs=out_specs,
        out_shape=out_shape,
        scratch_shapes=[
            pltpu.VMEM((NN, CQ, N), jnp.bfloat16),
            pltpu.VMEM((NN, CQ, 1), jnp.float32),
        ],
        compiler_params=pltpu.CompilerParams(
            dimension_semantics=("parallel", "arbitrary")),
    )(*inputs)
    return node_out, coors_out


# speedup vs baseline: 1.1283x; 1.1283x over previous
import jax
import jax.numpy as jnp
from jax.experimental import pallas as pl
from jax.experimental.pallas import tpu as pltpu

B, N, DIM, H, DH, NN = 4, 512, 256, 4, 32, 32
SCALE = 8.0
PB = 128
CQ = 128
NC = N // CQ


def _psi(x):
    return jnp.sign(x) * jnp.log1p(jnp.abs(x))


def _ln(x, g, b, eps=1e-5):
    m = jnp.mean(x, axis=-1, keepdims=True)
    v = jnp.mean((x - m) ** 2, axis=-1, keepdims=True)
    return (x - m) * jax.lax.rsqrt(v + eps) * g + b


def _silu(x):
    return x * jax.nn.sigmoid(x)


def _gelu(x):
    return 0.5 * x * (1.0 + jax.lax.erf(x * 0.7071067811865476))


def _lorentz_kernel(
    feats_ref, coors_ref, fc_ref, cc_ref, ct_ref,
    ng_ref, nb_ref, wqkv_ref, wout_ref, bout_ref,
    wc1_ref, wc2_ref, wg_ref, bg_ref, cns_ref, comb_ref,
    w0_ref, b0_ref, l0g_ref, l0b_ref,
    w1_ref, b1_ref, l1g_ref, l1b_ref,
    w2_ref, b2_ref, l2g_ref, l2b_ref,
    wqk_ref, bqk_ref, wv_ref, bv_ref,
    node_ref, cout_ref,
    ohs_ref, ds_ref,
):
    feats = feats_ref[0]
    coors = coors_ref[0]

    fn = _ln(feats, ng_ref[0], nb_ref[0])
    qkv = jnp.dot(fn, wqkv_ref[...], preferred_element_type=jnp.float32)

    r = jax.lax.broadcasted_iota(jnp.int32, (H * DH, H * DH), 0) // DH
    c = jax.lax.broadcasted_iota(jnp.int32, (H * DH, H * DH), 1) // DH
    S = (r == c).astype(jnp.float32)

    k = qkv[:, H * DH:2 * H * DH]
    v = qkv[:, 2 * H * DH:]
    ksq = jnp.dot(k * k, S, preferred_element_type=jnp.float32)
    kn = k / jnp.maximum(jnp.sqrt(ksq), 1e-12)
    kvc = jnp.concatenate([kn, v, coors], axis=1)

    fnc = _ln(fc_ref[0], ng_ref[0], nb_ref[0])
    q = jnp.dot(fnc, wqkv_ref[:, :H * DH], preferred_element_type=jnp.float32)
    qsq = jnp.dot(q * q, S, preferred_element_type=jnp.float32)
    qn = q / jnp.maximum(jnp.sqrt(qsq), 1e-12)

    coors_c = cc_ref[0]

    ct = ct_ref[0]
    raw = None
    for ci in range(4):
        dq = coors_c[:, ci:ci + 1] - ct[ci:ci + 1, :]
        sq = dq * dq
        raw = sq if ci == 0 else raw - sq
    dist = _psi(raw)

    iota_j = jax.lax.broadcasted_iota(jnp.int32, (CQ, N), 1)

    def body(t, d):
        dmin = jnp.min(d, axis=1, keepdims=True)
        ismin = d == dmin
        idx = jnp.min(jnp.where(ismin, iota_j, N), axis=1, keepdims=True)
        oh = iota_j == idx
        d = jnp.where(oh, jnp.float32(1e30), d)
        ohs_ref[t] = oh.astype(jnp.float32)
        ds_ref[t] = dmin
        return d

    jax.lax.fori_loop(0, NN, body, dist)

    ohf = ohs_ref[...].reshape(NN * CQ, N)
    g = jnp.dot(ohf, kvc, preferred_element_type=jnp.float32)
    kg = g[:, :H * DH].reshape(NN, CQ, H * DH)
    vg = g[:, H * DH:2 * H * DH].reshape(NN, CQ, H * DH)
    cg = g[:, 2 * H * DH:2 * H * DH + 4].reshape(NN, CQ, 4)

    w0v = w0_ref[...]
    b0v = b0_ref[...]
    g0 = l0g_ref[...]
    u = w0v - jnp.mean(w0v, axis=1, keepdims=True)
    ec = b0v - jnp.mean(b0v, axis=1, keepdims=True)
    v0 = jnp.mean(u * u, axis=1, keepdims=True).reshape(1, 1, 1)
    c0 = jnp.mean(u * ec, axis=1, keepdims=True).reshape(1, 1, 1)
    vb = jnp.mean(ec * ec, axis=1, keepdims=True).reshape(1, 1, 1)
    d0 = ds_ref[...]
    s = jax.lax.rsqrt(d0 * d0 * v0 + 2.0 * d0 * c0 + vb + 1e-5)
    x3 = (d0 * s) * (u * g0)[None] + s * (ec * g0)[None] + l0b_ref[...][None]
    x = _silu(x3.reshape(NN * CQ, PB))
    x = jnp.dot(x, w1_ref[...], preferred_element_type=jnp.float32) + b1_ref[...]
    x = _silu(_ln(x, l1g_ref[...], l1b_ref[...]))
    x = jnp.dot(x, w2_ref[...], preferred_element_type=jnp.float32) + b2_ref[...]
    x = _silu(_ln(x, l2g_ref[...], l2b_ref[...]))
    qk_pos = jnp.dot(x, wqk_ref[...], preferred_element_type=jnp.float32) + bqk_ref[...]
    vpos = jnp.dot(x, wv_ref[...], preferred_element_type=jnp.float32) + bv_ref[...]

    prod = (kg * qn[None]).reshape(NN * CQ, H * DH)
    qk_g = jnp.dot(prod, S, preferred_element_type=jnp.float32)
    qk = qk_g * SCALE + qk_pos
    qk3 = qk.reshape(NN, CQ, H * DH)

    mx = jnp.max(qk3, axis=0, keepdims=True)
    e = jnp.exp(qk3 - mx)
    attn = e / jnp.sum(e, axis=0, keepdims=True)

    vtot = vg + vpos.reshape(NN, CQ, H * DH)
    out_pre = jnp.sum(attn * vtot, axis=0)
    node = jnp.dot(out_pre, wout_ref[...], preferred_element_type=jnp.float32) + bout_ref[...]
    node_ref[0] = node

    h1 = _gelu(jnp.dot(qk, wc1_ref[...], preferred_element_type=jnp.float32))
    cw = jnp.dot(h1, wc2_ref[...], preferred_element_type=jnp.float32)
    cw3 = cw.reshape(NN, CQ, H * DH)
    cmx = jnp.max(cw3, axis=0, keepdims=True)
    ce = jnp.exp(cw3 - cmx)
    ca = ce / jnp.sum(ce, axis=0, keepdims=True)
    gate = jnp.tanh(
        jnp.dot(qk, wg_ref[...], preferred_element_type=jnp.float32) + bg_ref[...]
    ).reshape(NN, CQ, H * DH)
    w = jnp.sum(ca * gate * comb_ref[...], axis=2, keepdims=True)
    rel = (coors_c[None] - cg) * cns_ref[0, 0]
    cout_ref[0] = jnp.sum(w * rel, axis=0)


def kernel(feats, coors, params):
    p = params
    coors_t = jnp.swapaxes(coors, 1, 2)

    def rep(x, axis):
        return jnp.repeat(x, DH, axis=axis)

    wc1 = rep(p['W_c1'], 0) / DH
    wc2 = rep(p['W_c2'], 1)
    wg = rep(rep(p['W_gate'], 0), 1) / DH
    bg = rep(p['b_gate'][None], 1)
    comb = rep(p['coors_combine'][None], 1) / DH
    wqk = rep(p['pb_Wqk'], 1)
    bqk = rep(p['pb_bqk'][None], 1)

    def row(x):
        return x[None].astype(jnp.float32)

    inputs = [
        feats, coors, feats, coors, coors_t,
        row(p['norm_gamma']), row(p['norm_beta']),
        p['W_qkv'], p['W_out'], row(p['b_out']),
        wc1, wc2, wg, bg,
        p['coors_norm_scale'][None], comb,
        p['pb_W0'], row(p['pb_b0']), row(p['pb_ln0_g']), row(p['pb_ln0_b']),
        p['pb_W1'], row(p['pb_b1']), row(p['pb_ln1_g']), row(p['pb_ln1_b']),
        p['pb_W2'], row(p['pb_b2']), row(p['pb_ln2_g']), row(p['pb_ln2_b']),
        wqk, bqk,
        p['pb_Wv'], row(p['pb_bv']),
    ]

    def batch_spec(shape):
        return pl.BlockSpec((1,) + shape, lambda b, c: (b, 0, 0))

    def full_spec(x):
        nd = x.ndim
        return pl.BlockSpec(x.shape, lambda b, c, _n=nd: (0,) * _n)

    def chunk_spec(lanes):
        return pl.BlockSpec((1, CQ, lanes), lambda b, c: (b, c, 0))

    in_specs = [
        batch_spec((N, DIM)), batch_spec((N, 4)),
        chunk_spec(DIM), chunk_spec(4),
        batch_spec((4, N)),
    ] + [full_spec(x) for x in inputs[5:]]

    out_shape = [
        jax.ShapeDtypeStruct((B, N, DIM), jnp.float32),
        jax.ShapeDtypeStruct((B, N, 4), jnp.float32),
    ]
    out_specs = [
        pl.BlockSpec((1, CQ, DIM), lambda b, c: (b, c, 0)),
        pl.BlockSpec((1, CQ, 4), lambda b, c: (b, c, 0)),
    ]

    node_out, coors_out = pl.pallas_call(
        _lorentz_kernel,
        grid=(B, NC),
        in_specs=in_specs,
        out_specs=out_specs,
        out_shape=out_shape,
        scratch_shapes=[
            pltpu.VMEM((NN, CQ, N), jnp.float32),
            pltpu.VMEM((NN, CQ, 1), jnp.float32),
        ],
        compiler_params=pltpu.CompilerParams(
            dimension_semantics=("parallel", "arbitrary")),
    )(*inputs)
    return node_out, coors_out


# speedup vs baseline: 1.4326x; 1.2697x over previous
import jax
import jax.numpy as jnp
from jax.experimental import pallas as pl
from jax.experimental.pallas import tpu as pltpu

B, N, DIM, H, DH, NN = 4, 512, 256, 4, 32, 32
SCALE = 8.0
PB = 128
CQ = 256
NC = N // CQ


def _psi(x):
    return jnp.sign(x) * jnp.log1p(jnp.abs(x))


def _ln(x, g, b, eps=1e-5):
    m = jnp.mean(x, axis=-1, keepdims=True)
    v = jnp.mean((x - m) ** 2, axis=-1, keepdims=True)
    return (x - m) * jax.lax.rsqrt(v + eps) * g + b


def _silu(x):
    return x * jax.nn.sigmoid(x)


def _gelu(x):
    return 0.5 * x * (1.0 + jax.lax.erf(x * 0.7071067811865476))


def _lorentz_kernel(
    feats_ref, coors_ref, fc_ref, cc_ref, ct_ref,
    ng_ref, nb_ref, wqkv_ref, wout_ref, bout_ref,
    wc1_ref, wc2_ref, wg_ref, bg_ref, cns_ref, comb_ref,
    w0_ref, b0_ref, l0g_ref, l0b_ref,
    w1_ref, b1_ref, l1g_ref, l1b_ref,
    w2_ref, b2_ref, l2g_ref, l2b_ref,
    wqk_ref, bqk_ref, wv_ref, bv_ref,
    node_ref, cout_ref,
    ohs_ref, ds_ref,
):
    feats = feats_ref[0]
    coors = coors_ref[0]

    fn = _ln(feats, ng_ref[0], nb_ref[0])
    qkv = jnp.dot(fn, wqkv_ref[...], preferred_element_type=jnp.float32)

    r = jax.lax.broadcasted_iota(jnp.int32, (H * DH, H * DH), 0) // DH
    c = jax.lax.broadcasted_iota(jnp.int32, (H * DH, H * DH), 1) // DH
    S = (r == c).astype(jnp.float32)

    k = qkv[:, H * DH:2 * H * DH]
    v = qkv[:, 2 * H * DH:]
    ksq = jnp.dot(k * k, S, preferred_element_type=jnp.float32)
    kn = k / jnp.maximum(jnp.sqrt(ksq), 1e-12)
    kvc = jnp.concatenate([kn, v, coors], axis=1)

    fnc = _ln(fc_ref[0], ng_ref[0], nb_ref[0])
    q = jnp.dot(fnc, wqkv_ref[:, :H * DH], preferred_element_type=jnp.float32)
    qsq = jnp.dot(q * q, S, preferred_element_type=jnp.float32)
    qn = q / jnp.maximum(jnp.sqrt(qsq), 1e-12)

    coors_c = cc_ref[0]

    ct = ct_ref[0]
    raw = None
    for ci in range(4):
        dq = coors_c[:, ci:ci + 1] - ct[ci:ci + 1, :]
        sq = dq * dq
        raw = sq if ci == 0 else raw - sq
    dist = _psi(raw)

    iota_j = jax.lax.broadcasted_iota(jnp.int32, (CQ, N), 1)

    def body(t, d):
        dmin = jnp.min(d, axis=1, keepdims=True)
        ismin = d == dmin
        idx = jnp.min(jnp.where(ismin, iota_j, N), axis=1, keepdims=True)
        oh = iota_j == idx
        d = jnp.where(oh, jnp.float32(1e30), d)
        ohs_ref[t] = oh.astype(jnp.float32)
        ds_ref[t] = dmin
        return d

    jax.lax.fori_loop(0, NN, body, dist)

    ohf = ohs_ref[...].reshape(NN * CQ, N)
    g = jnp.dot(ohf, kvc, preferred_element_type=jnp.float32)
    kg = g[:, :H * DH].reshape(NN, CQ, H * DH)
    vg = g[:, H * DH:2 * H * DH].reshape(NN, CQ, H * DH)
    cg = g[:, 2 * H * DH:2 * H * DH + 4].reshape(NN, CQ, 4)

    w0v = w0_ref[...]
    b0v = b0_ref[...]
    g0 = l0g_ref[...]
    u = w0v - jnp.mean(w0v, axis=1, keepdims=True)
    ec = b0v - jnp.mean(b0v, axis=1, keepdims=True)
    v0 = jnp.mean(u * u, axis=1, keepdims=True).reshape(1, 1, 1)
    c0 = jnp.mean(u * ec, axis=1, keepdims=True).reshape(1, 1, 1)
    vb = jnp.mean(ec * ec, axis=1, keepdims=True).reshape(1, 1, 1)
    d0 = ds_ref[...]
    s = jax.lax.rsqrt(d0 * d0 * v0 + 2.0 * d0 * c0 + vb + 1e-5)
    x3 = (d0 * s) * (u * g0)[None] + s * (ec * g0)[None] + l0b_ref[...][None]
    x = _silu(x3.reshape(NN * CQ, PB))
    x = jnp.dot(x, w1_ref[...], preferred_element_type=jnp.float32) + b1_ref[...]
    x = _silu(_ln(x, l1g_ref[...], l1b_ref[...]))
    x = jnp.dot(x, w2_ref[...], preferred_element_type=jnp.float32) + b2_ref[...]
    x = _silu(_ln(x, l2g_ref[...], l2b_ref[...]))
    qk_pos = jnp.dot(x, wqk_ref[...], preferred_element_type=jnp.float32) + bqk_ref[...]
    vpos = jnp.dot(x, wv_ref[...], preferred_element_type=jnp.float32) + bv_ref[...]

    prod = (kg * qn[None]).reshape(NN * CQ, H * DH)
    qk_g = jnp.dot(prod, S, preferred_element_type=jnp.float32)
    qk = qk_g * SCALE + qk_pos
    qk3 = qk.reshape(NN, CQ, H * DH)

    mx = jnp.max(qk3, axis=0, keepdims=True)
    e = jnp.exp(qk3 - mx)
    attn = e / jnp.sum(e, axis=0, keepdims=True)

    vtot = vg + vpos.reshape(NN, CQ, H * DH)
    out_pre = jnp.sum(attn * vtot, axis=0)
    node = jnp.dot(out_pre, wout_ref[...], preferred_element_type=jnp.float32) + bout_ref[...]
    node_ref[0] = node

    h1 = _gelu(jnp.dot(qk, wc1_ref[...], preferred_element_type=jnp.float32))
    cw = jnp.dot(h1, wc2_ref[...], preferred_element_type=jnp.float32)
    cw3 = cw.reshape(NN, CQ, H * DH)
    cmx = jnp.max(cw3, axis=0, keepdims=True)
    ce = jnp.exp(cw3 - cmx)
    ca = ce / jnp.sum(ce, axis=0, keepdims=True)
    gate = jnp.tanh(
        jnp.dot(qk, wg_ref[...], preferred_element_type=jnp.float32) + bg_ref[...]
    ).reshape(NN, CQ, H * DH)
    w = jnp.sum(ca * gate * comb_ref[...], axis=2, keepdims=True)
    rel = (coors_c[None] - cg) * cns_ref[0, 0]
    cout_ref[0] = jnp.sum(w * rel, axis=0)


def kernel(feats, coors, params):
    p = params
    coors_t = jnp.swapaxes(coors, 1, 2)

    def rep(x, axis):
        return jnp.repeat(x, DH, axis=axis)

    wc1 = rep(p['W_c1'], 0) / DH
    wc2 = rep(p['W_c2'], 1)
    wg = rep(rep(p['W_gate'], 0), 1) / DH
    bg = rep(p['b_gate'][None], 1)
    comb = rep(p['coors_combine'][None], 1) / DH
    wqk = rep(p['pb_Wqk'], 1)
    bqk = rep(p['pb_bqk'][None], 1)

    def row(x):
        return x[None].astype(jnp.float32)

    inputs = [
        feats, coors, feats, coors, coors_t,
        row(p['norm_gamma']), row(p['norm_beta']),
        p['W_qkv'], p['W_out'], row(p['b_out']),
        wc1, wc2, wg, bg,
        p['coors_norm_scale'][None], comb,
        p['pb_W0'], row(p['pb_b0']), row(p['pb_ln0_g']), row(p['pb_ln0_b']),
        p['pb_W1'], row(p['pb_b1']), row(p['pb_ln1_g']), row(p['pb_ln1_b']),
        p['pb_W2'], row(p['pb_b2']), row(p['pb_ln2_g']), row(p['pb_ln2_b']),
        wqk, bqk,
        p['pb_Wv'], row(p['pb_bv']),
    ]

    def batch_spec(shape):
        return pl.BlockSpec((1,) + shape, lambda b, c: (b, 0, 0))

    def full_spec(x):
        nd = x.ndim
        return pl.BlockSpec(x.shape, lambda b, c, _n=nd: (0,) * _n)

    def chunk_spec(lanes):
        return pl.BlockSpec((1, CQ, lanes), lambda b, c: (b, c, 0))

    in_specs = [
        batch_spec((N, DIM)), batch_spec((N, 4)),
        chunk_spec(DIM), chunk_spec(4),
        batch_spec((4, N)),
    ] + [full_spec(x) for x in inputs[5:]]

    out_shape = [
        jax.ShapeDtypeStruct((B, N, DIM), jnp.float32),
        jax.ShapeDtypeStruct((B, N, 4), jnp.float32),
    ]
    out_specs = [
        pl.BlockSpec((1, CQ, DIM), lambda b, c: (b, c, 0)),
        pl.BlockSpec((1, CQ, 4), lambda b, c: (b, c, 0)),
    ]

    node_out, coors_out = pl.pallas_call(
        _lorentz_kernel,
        grid=(B, NC),
        in_specs=in_specs,
        out_specs=out_specs,
        out_shape=out_shape,
        scratch_shapes=[
            pltpu.VMEM((NN, CQ, N), jnp.float32),
            pltpu.VMEM((NN, CQ, 1), jnp.float32),
        ],
        compiler_params=pltpu.CompilerParams(
            dimension_semantics=("parallel", "arbitrary")),
    )(*inputs)
    return node_out, coors_out


# speedup vs baseline: 1.5222x; 1.0625x over previous
import jax
import jax.numpy as jnp
from jax.experimental import pallas as pl
from jax.experimental.pallas import tpu as pltpu

B, N, DIM, H, DH, NN = 4, 512, 256, 4, 32, 32
SCALE = 8.0
PB = 128
CQ = 256
NC = N // CQ


def _psi(x):
    return jnp.sign(x) * jnp.log1p(jnp.abs(x))


def _ln(x, g, b, eps=1e-5):
    m = jnp.mean(x, axis=-1, keepdims=True)
    v = jnp.mean((x - m) ** 2, axis=-1, keepdims=True)
    return (x - m) * jax.lax.rsqrt(v + eps) * g + b


def _silu(x):
    return x * jax.nn.sigmoid(x)


def _gelu(x):
    return 0.5 * x * (1.0 + jax.lax.erf(x * 0.7071067811865476))


def _lorentz_kernel(
    feats_ref, coors_ref, fc_ref, cc_ref, ct_ref,
    ng_ref, nb_ref, wqkv_ref, wout_ref, bout_ref,
    wc1_ref, wc2_ref, wg_ref, bg_ref, cns_ref, comb_ref,
    w0_ref, b0_ref, l0g_ref, l0b_ref,
    w1_ref, b1_ref, l1g_ref, l1b_ref,
    w2_ref, b2_ref, l2g_ref, l2b_ref,
    wqk_ref, bqk_ref, wv_ref, bv_ref,
    node_ref, cout_ref,
    ohs_ref, ds_ref,
):
    feats = feats_ref[0]
    coors = coors_ref[0]

    fn = _ln(feats, ng_ref[0], nb_ref[0])
    qkv = jnp.dot(fn, wqkv_ref[...], preferred_element_type=jnp.float32)

    r = jax.lax.broadcasted_iota(jnp.int32, (H * DH, H * DH), 0) // DH
    c = jax.lax.broadcasted_iota(jnp.int32, (H * DH, H * DH), 1) // DH
    S = (r == c).astype(jnp.float32)

    k = qkv[:, H * DH:2 * H * DH]
    v = qkv[:, 2 * H * DH:]
    ksq = jnp.dot(k * k, S, preferred_element_type=jnp.float32)
    kn = k / jnp.maximum(jnp.sqrt(ksq), 1e-12)
    kvc = jnp.concatenate([kn, v, coors], axis=1)

    fnc = _ln(fc_ref[0], ng_ref[0], nb_ref[0])
    q = jnp.dot(fnc, wqkv_ref[:, :H * DH], preferred_element_type=jnp.float32)
    qsq = jnp.dot(q * q, S, preferred_element_type=jnp.float32)
    qn = q / jnp.maximum(jnp.sqrt(qsq), 1e-12)

    coors_c = cc_ref[0]

    ct = ct_ref[0]
    raw = None
    for ci in range(4):
        dq = coors_c[:, ci:ci + 1] - ct[ci:ci + 1, :]
        sq = dq * dq
        raw = sq if ci == 0 else raw - sq
    dist = _psi(raw)

    iota_j = jax.lax.broadcasted_iota(jnp.int32, (CQ, N), 1)

    def select_one(t, d):
        dmin = jnp.min(d, axis=1, keepdims=True)
        ismin = d == dmin
        idx = jnp.min(jnp.where(ismin, iota_j, N), axis=1, keepdims=True)
        oh = iota_j == idx
        d = jnp.where(oh, jnp.float32(1e30), d)
        ohs_ref[t] = oh.astype(jnp.float32)
        ds_ref[t] = dmin
        return d

    def body(t2, d):
        d = select_one(2 * t2, d)
        d = select_one(2 * t2 + 1, d)
        return d

    jax.lax.fori_loop(0, NN // 2, body, dist)

    ohf = ohs_ref[...].reshape(NN * CQ, N)
    g = jnp.dot(ohf, kvc, preferred_element_type=jnp.float32)
    kg = g[:, :H * DH].reshape(NN, CQ, H * DH)
    vg = g[:, H * DH:2 * H * DH].reshape(NN, CQ, H * DH)
    cg = g[:, 2 * H * DH:2 * H * DH + 4].reshape(NN, CQ, 4)

    w0v = w0_ref[...]
    b0v = b0_ref[...]
    g0 = l0g_ref[...]
    u = w0v - jnp.mean(w0v, axis=1, keepdims=True)
    ec = b0v - jnp.mean(b0v, axis=1, keepdims=True)
    v0 = jnp.mean(u * u, axis=1, keepdims=True).reshape(1, 1, 1)
    c0 = jnp.mean(u * ec, axis=1, keepdims=True).reshape(1, 1, 1)
    vb = jnp.mean(ec * ec, axis=1, keepdims=True).reshape(1, 1, 1)
    d0 = ds_ref[...]
    s = jax.lax.rsqrt(d0 * d0 * v0 + 2.0 * d0 * c0 + vb + 1e-5)
    x3 = (d0 * s) * (u * g0)[None] + s * (ec * g0)[None] + l0b_ref[...][None]
    x = _silu(x3.reshape(NN * CQ, PB))
    x = jnp.dot(x, w1_ref[...], preferred_element_type=jnp.float32) + b1_ref[...]
    x = _silu(_ln(x, l1g_ref[...], l1b_ref[...]))
    x = jnp.dot(x, w2_ref[...], preferred_element_type=jnp.float32) + b2_ref[...]
    x = _silu(_ln(x, l2g_ref[...], l2b_ref[...]))
    qk_pos = jnp.dot(x, wqk_ref[...], preferred_element_type=jnp.float32) + bqk_ref[...]
    vpos = jnp.dot(x, wv_ref[...], preferred_element_type=jnp.float32) + bv_ref[...]

    prod = (kg * qn[None]).reshape(NN * CQ, H * DH)
    qk_g = jnp.dot(prod, S, preferred_element_type=jnp.float32)
    qk = qk_g * SCALE + qk_pos
    qk3 = qk.reshape(NN, CQ, H * DH)

    mx = jnp.max(qk3, axis=0, keepdims=True)
    e = jnp.exp(qk3 - mx)
    attn = e / jnp.sum(e, axis=0, keepdims=True)

    vtot = vg + vpos.reshape(NN, CQ, H * DH)
    out_pre = jnp.sum(attn * vtot, axis=0)
    node = jnp.dot(out_pre, wout_ref[...], preferred_element_type=jnp.float32) + bout_ref[...]
    node_ref[0] = node

    h1 = _gelu(jnp.dot(qk, wc1_ref[...], preferred_element_type=jnp.float32))
    cw = jnp.dot(h1, wc2_ref[...], preferred_element_type=jnp.float32)
    cw3 = cw.reshape(NN, CQ, H * DH)
    cmx = jnp.max(cw3, axis=0, keepdims=True)
    ce = jnp.exp(cw3 - cmx)
    ca = ce / jnp.sum(ce, axis=0, keepdims=True)
    gate = jnp.tanh(
        jnp.dot(qk, wg_ref[...], preferred_element_type=jnp.float32) + bg_ref[...]
    ).reshape(NN, CQ, H * DH)
    w = jnp.sum(ca * gate * comb_ref[...], axis=2, keepdims=True)
    rel = (coors_c[None] - cg) * cns_ref[0, 0]
    cout_ref[0] = jnp.sum(w * rel, axis=0)


def kernel(feats, coors, params):
    p = params
    coors_t = jnp.swapaxes(coors, 1, 2)

    def rep(x, axis):
        return jnp.repeat(x, DH, axis=axis)

    wc1 = rep(p['W_c1'], 0) / DH
    wc2 = rep(p['W_c2'], 1)
    wg = rep(rep(p['W_gate'], 0), 1) / DH
    bg = rep(p['b_gate'][None], 1)
    comb = rep(p['coors_combine'][None], 1) / DH
    wqk = rep(p['pb_Wqk'], 1)
    bqk = rep(p['pb_bqk'][None], 1)

    def row(x):
        return x[None].astype(jnp.float32)

    inputs = [
        feats, coors, feats, coors, coors_t,
        row(p['norm_gamma']), row(p['norm_beta']),
        p['W_qkv'], p['W_out'], row(p['b_out']),
        wc1, wc2, wg, bg,
        p['coors_norm_scale'][None], comb,
        p['pb_W0'], row(p['pb_b0']), row(p['pb_ln0_g']), row(p['pb_ln0_b']),
        p['pb_W1'], row(p['pb_b1']), row(p['pb_ln1_g']), row(p['pb_ln1_b']),
        p['pb_W2'], row(p['pb_b2']), row(p['pb_ln2_g']), row(p['pb_ln2_b']),
        wqk, bqk,
        p['pb_Wv'], row(p['pb_bv']),
    ]

    def batch_spec(shape):
        return pl.BlockSpec((1,) + shape, lambda b, c: (b, 0, 0))

    def full_spec(x):
        nd = x.ndim
        return pl.BlockSpec(x.shape, lambda b, c, _n=nd: (0,) * _n)

    def chunk_spec(lanes):
        return pl.BlockSpec((1, CQ, lanes), lambda b, c: (b, c, 0))

    in_specs = [
        batch_spec((N, DIM)), batch_spec((N, 4)),
        chunk_spec(DIM), chunk_spec(4),
        batch_spec((4, N)),
    ] + [full_spec(x) for x in inputs[5:]]

    out_shape = [
        jax.ShapeDtypeStruct((B, N, DIM), jnp.float32),
        jax.ShapeDtypeStruct((B, N, 4), jnp.float32),
    ]
    out_specs = [
        pl.BlockSpec((1, CQ, DIM), lambda b, c: (b, c, 0)),
        pl.BlockSpec((1, CQ, 4), lambda b, c: (b, c, 0)),
    ]

    node_out, coors_out = pl.pallas_call(
        _lorentz_kernel,
        grid=(B, NC),
        in_specs=in_specs,
        out_specs=out_specs,
        out_shape=out_shape,
        scratch_shapes=[
            pltpu.VMEM((NN, CQ, N), jnp.float32),
            pltpu.VMEM((NN, CQ, 1), jnp.float32),
        ],
        compiler_params=pltpu.CompilerParams(
            dimension_semantics=("parallel", "arbitrary")),
    )(*inputs)
    return node_out, coors_out


# speedup vs baseline: 1.5762x; 1.0355x over previous
import jax
import jax.numpy as jnp
from jax.experimental import pallas as pl
from jax.experimental.pallas import tpu as pltpu

B, N, DIM, H, DH, NN = 4, 512, 256, 4, 32, 32
SCALE = 8.0
PB = 128
CQ = 256
NC = N // CQ


def _psi(x):
    return jnp.sign(x) * jnp.log1p(jnp.abs(x))


def _ln(x, g, b, eps=1e-5):
    m = jnp.mean(x, axis=-1, keepdims=True)
    v = jnp.mean((x - m) ** 2, axis=-1, keepdims=True)
    return (x - m) * jax.lax.rsqrt(v + eps) * g + b


def _silu(x):
    return x * jax.nn.sigmoid(x)


def _gelu(x):
    return 0.5 * x * (1.0 + jax.lax.erf(x * 0.7071067811865476))


def _lorentz_kernel(
    feats_ref, coors_ref, fc_ref, cc_ref, ct_ref,
    ng_ref, nb_ref, wqkv_ref, wout_ref, bout_ref,
    wc1_ref, wc2_ref, wg_ref, bg_ref, cns_ref, comb_ref,
    w0_ref, b0_ref, l0g_ref, l0b_ref,
    w1_ref, b1_ref, l1g_ref, l1b_ref,
    w2_ref, b2_ref, l2g_ref, l2b_ref,
    wqk_ref, bqk_ref, wv_ref, bv_ref,
    node_ref, cout_ref,
    ohs_ref, ds_ref,
):
    feats = feats_ref[0]
    coors = coors_ref[0]

    fn = _ln(feats, ng_ref[0], nb_ref[0])
    qkv = jnp.dot(fn, wqkv_ref[...], preferred_element_type=jnp.float32)

    r = jax.lax.broadcasted_iota(jnp.int32, (H * DH, H * DH), 0) // DH
    c = jax.lax.broadcasted_iota(jnp.int32, (H * DH, H * DH), 1) // DH
    S = (r == c).astype(jnp.float32)

    k = qkv[:, H * DH:2 * H * DH]
    v = qkv[:, 2 * H * DH:]
    ksq = jnp.dot(k * k, S, preferred_element_type=jnp.float32)
    kn = k / jnp.maximum(jnp.sqrt(ksq), 1e-12)
    kvc = jnp.concatenate([kn, v, coors], axis=1)

    fnc = _ln(fc_ref[0], ng_ref[0], nb_ref[0])
    q = jnp.dot(fnc, wqkv_ref[:, :H * DH], preferred_element_type=jnp.float32)
    qsq = jnp.dot(q * q, S, preferred_element_type=jnp.float32)
    qn = q / jnp.maximum(jnp.sqrt(qsq), 1e-12)

    coors_c = cc_ref[0]

    ct = ct_ref[0]
    raw = None
    for ci in range(4):
        dq = coors_c[:, ci:ci + 1] - ct[ci:ci + 1, :]
        sq = dq * dq
        raw = sq if ci == 0 else raw - sq
    dist = _psi(raw)

    iota_j = jax.lax.broadcasted_iota(jnp.int32, (CQ, N), 1)

    def select_one(t, d):
        dmin = jnp.min(d, axis=1, keepdims=True)
        ismin = d == dmin
        idx = jnp.min(jnp.where(ismin, iota_j, N), axis=1, keepdims=True)
        oh = iota_j == idx
        d = jnp.where(oh, jnp.float32(1e30), d)
        ohs_ref[t] = oh.astype(jnp.float32)
        ds_ref[t] = dmin
        return d

    def body(t4, d):
        for u in range(4):
            d = select_one(4 * t4 + u, d)
        return d

    jax.lax.fori_loop(0, NN // 4, body, dist)

    ohf = ohs_ref[...].reshape(NN * CQ, N)
    g = jnp.dot(ohf, kvc, preferred_element_type=jnp.float32)
    kg = g[:, :H * DH].reshape(NN, CQ, H * DH)
    vg = g[:, H * DH:2 * H * DH].reshape(NN, CQ, H * DH)
    cg = g[:, 2 * H * DH:2 * H * DH + 4].reshape(NN, CQ, 4)

    w0v = w0_ref[...]
    b0v = b0_ref[...]
    g0 = l0g_ref[...]
    u = w0v - jnp.mean(w0v, axis=1, keepdims=True)
    ec = b0v - jnp.mean(b0v, axis=1, keepdims=True)
    v0 = jnp.mean(u * u, axis=1, keepdims=True).reshape(1, 1, 1)
    c0 = jnp.mean(u * ec, axis=1, keepdims=True).reshape(1, 1, 1)
    vb = jnp.mean(ec * ec, axis=1, keepdims=True).reshape(1, 1, 1)
    d0 = ds_ref[...]
    s = jax.lax.rsqrt(d0 * d0 * v0 + 2.0 * d0 * c0 + vb + 1e-5)
    x3 = (d0 * s) * (u * g0)[None] + s * (ec * g0)[None] + l0b_ref[...][None]
    x = _silu(x3.reshape(NN * CQ, PB))
    x = jnp.dot(x, w1_ref[...], preferred_element_type=jnp.float32) + b1_ref[...]
    x = _silu(_ln(x, l1g_ref[...], l1b_ref[...]))
    x = jnp.dot(x, w2_ref[...], preferred_element_type=jnp.float32) + b2_ref[...]
    x = _silu(_ln(x, l2g_ref[...], l2b_ref[...]))
    qk_pos = jnp.dot(x, wqk_ref[...], preferred_element_type=jnp.float32) + bqk_ref[...]
    vpos = jnp.dot(x, wv_ref[...], preferred_element_type=jnp.float32) + bv_ref[...]

    prod = (kg * qn[None]).reshape(NN * CQ, H * DH)
    qk_g = jnp.dot(prod, S, preferred_element_type=jnp.float32)
    qk = qk_g * SCALE + qk_pos
    qk3 = qk.reshape(NN, CQ, H * DH)

    mx = jnp.max(qk3, axis=0, keepdims=True)
    e = jnp.exp(qk3 - mx)
    attn = e / jnp.sum(e, axis=0, keepdims=True)

    vtot = vg + vpos.reshape(NN, CQ, H * DH)
    out_pre = jnp.sum(attn * vtot, axis=0)
    node = jnp.dot(out_pre, wout_ref[...], preferred_element_type=jnp.float32) + bout_ref[...]
    node_ref[0] = node

    h1 = _gelu(jnp.dot(qk, wc1_ref[...], preferred_element_type=jnp.float32))
    cw = jnp.dot(h1, wc2_ref[...], preferred_element_type=jnp.float32)
    cw3 = cw.reshape(NN, CQ, H * DH)
    cmx = jnp.max(cw3, axis=0, keepdims=True)
    ce = jnp.exp(cw3 - cmx)
    ca = ce / jnp.sum(ce, axis=0, keepdims=True)
    gate = jnp.tanh(
        jnp.dot(qk, wg_ref[...], preferred_element_type=jnp.float32) + bg_ref[...]
    ).reshape(NN, CQ, H * DH)
    w = jnp.sum(ca * gate * comb_ref[...], axis=2, keepdims=True)
    rel = (coors_c[None] - cg) * cns_ref[0, 0]
    cout_ref[0] = jnp.sum(w * rel, axis=0)


def kernel(feats, coors, params):
    p = params
    coors_t = jnp.swapaxes(coors, 1, 2)

    def rep(x, axis):
        return jnp.repeat(x, DH, axis=axis)

    wc1 = rep(p['W_c1'], 0) / DH
    wc2 = rep(p['W_c2'], 1)
    wg = rep(rep(p['W_gate'], 0), 1) / DH
    bg = rep(p['b_gate'][None], 1)
    comb = rep(p['coors_combine'][None], 1) / DH
    wqk = rep(p['pb_Wqk'], 1)
    bqk = rep(p['pb_bqk'][None], 1)

    def row(x):
        return x[None].astype(jnp.float32)

    inputs = [
        feats, coors, feats, coors, coors_t,
        row(p['norm_gamma']), row(p['norm_beta']),
        p['W_qkv'], p['W_out'], row(p['b_out']),
        wc1, wc2, wg, bg,
        p['coors_norm_scale'][None], comb,
        p['pb_W0'], row(p['pb_b0']), row(p['pb_ln0_g']), row(p['pb_ln0_b']),
        p['pb_W1'], row(p['pb_b1']), row(p['pb_ln1_g']), row(p['pb_ln1_b']),
        p['pb_W2'], row(p['pb_b2']), row(p['pb_ln2_g']), row(p['pb_ln2_b']),
        wqk, bqk,
        p['pb_Wv'], row(p['pb_bv']),
    ]

    def batch_spec(shape):
        return pl.BlockSpec((1,) + shape, lambda b, c: (b, 0, 0))

    def full_spec(x):
        nd = x.ndim
        return pl.BlockSpec(x.shape, lambda b, c, _n=nd: (0,) * _n)

    def chunk_spec(lanes):
        return pl.BlockSpec((1, CQ, lanes), lambda b, c: (b, c, 0))

    in_specs = [
        batch_spec((N, DIM)), batch_spec((N, 4)),
        chunk_spec(DIM), chunk_spec(4),
        batch_spec((4, N)),
    ] + [full_spec(x) for x in inputs[5:]]

    out_shape = [
        jax.ShapeDtypeStruct((B, N, DIM), jnp.float32),
        jax.ShapeDtypeStruct((B, N, 4), jnp.float32),
    ]
    out_specs = [
        pl.BlockSpec((1, CQ, DIM), lambda b, c: (b, c, 0)),
        pl.BlockSpec((1, CQ, 4), lambda b, c: (b, c, 0)),
    ]

    node_out, coors_out = pl.pallas_call(
        _lorentz_kernel,
        grid=(B, NC),
        in_specs=in_specs,
        out_specs=out_specs,
        out_shape=out_shape,
        scratch_shapes=[
            pltpu.VMEM((NN, CQ, N), jnp.float32),
            pltpu.VMEM((NN, CQ, 1), jnp.float32),
        ],
        compiler_params=pltpu.CompilerParams(
            dimension_semantics=("parallel", "arbitrary")),
    )(*inputs)
    return node_out, coors_out
